# Initial kernel scaffold; baseline (speedup 1.0000x reference)
import jax
import jax.numpy as jnp
from jax.experimental import pallas as pl


def kernel(img, img_shapes, queries, reference_points, W_img, b_img, W_q, b_q, W_out, b_out):
    raise NotImplementedError("write your pallas kernel here")


# speedup vs baseline: 45.9945x; 45.9945x over previous
import functools

import numpy as np
import jax
import jax.numpy as jnp
from jax import lax
from jax.experimental import pallas as pl
from jax.experimental.pallas import tpu as pltpu
from jax.experimental.pallas import tpu_sc as plsc

B = 4
N = 900
I_TOT = 5440
H = 8
L = 4
P = 4
HEAD_DIM = 32
EMB = 256
QOUT = 384
NQ = B * N
ROWS_Q = 4 * H * L * P

NW = 32
Q_PER_W = -(-NQ // NW)

_LEVEL_W = (64.0, 32.0, 16.0, 8.0)
_LEVEL_START = (0, 4096, 5120, 5376)


def _xproj_body(img_ref, w_ref, b_ref, o_ref):
    acc = jax.lax.dot_general(
        img_ref[0], w_ref[...], (((1,), (0,)), ((), ())),
        preferred_element_type=jnp.float32,
        precision=jax.lax.Precision.HIGHEST)
    o_ref[...] = (acc + b_ref[...])[None]


def _xproj(img, w_t, bias2d):
    tile = 680
    return pl.pallas_call(
        _xproj_body,
        grid=(B, I_TOT // tile),
        in_specs=[
            pl.BlockSpec((1, tile, EMB), lambda b, i: (b, i, 0)),
            pl.BlockSpec((EMB, EMB), lambda b, i: (0, 0)),
            pl.BlockSpec((1, EMB), lambda b, i: (0, 0)),
        ],
        out_specs=pl.BlockSpec((1, tile, EMB), lambda b, i: (b, i, 0)),
        out_shape=jax.ShapeDtypeStruct((B, I_TOT, EMB), jnp.float32),
    )(img, w_t, bias2d)


def _prep_body(qT_ref, rpT_ref, w_ref, b_ref, idx_ref, wt_ref):
    b = pl.program_id(0)
    q = jax.lax.dot_general(
        w_ref[...], qT_ref[0], (((1,), (0,)), ((), ())),
        preferred_element_type=jnp.float32,
        precision=jax.lax.Precision.HIGHEST) + b_ref[...]
    xo = q[0:128]
    yo = q[128:256]
    lg = q[256:384]
    attn_blocks = []
    for h in range(H):
        blk = lg[h * 16:(h + 1) * 16]
        m = jnp.max(blk, axis=0, keepdims=True)
        e = jnp.exp(blk - m)
        s = jnp.sum(e, axis=0, keepdims=True)
        attn_blocks.append(e / s)
    attn = jnp.concatenate(attn_blocks, axis=0)

    rx = rpT_ref[0, 0:1]
    ry = rpT_ref[0, 1:2]

    r_iota = lax.broadcasted_iota(jnp.int32, (128, 900), 0)
    li = (r_iota // 4) % 4
    h_idx = r_iota // 16
    wf = jnp.where(li == 0, _LEVEL_W[0],
                   jnp.where(li == 1, _LEVEL_W[1],
                             jnp.where(li == 2, _LEVEL_W[2], _LEVEL_W[3])))
    wi = jnp.where(li == 0, 64, jnp.where(li == 1, 32,
                                          jnp.where(li == 2, 16, 8)))
    start = jnp.where(li == 0, _LEVEL_START[0],
                      jnp.where(li == 1, _LEVEL_START[1],
                                jnp.where(li == 2, _LEVEL_START[2],
                                          _LEVEL_START[3])))

    ix = rx * wf + xo - 0.5
    iy = ry * wf + yo - 0.5
    x0 = jnp.floor(ix)
    y0 = jnp.floor(iy)
    wx1 = ix - x0
    wx0 = 1.0 - wx1
    wy1 = iy - y0
    wy0 = 1.0 - wy1

    base = b * (I_TOT * H)
    for ci, (cx, cy) in enumerate(((0, 0), (1, 0), (0, 1), (1, 1))):
        xf = x0 + cx
        yf = y0 + cy
        wxy = (wx1 if cx else wx0) * (wy1 if cy else wy0)
        valid = ((xf >= 0) & (xf <= wf - 1) & (yf >= 0) & (yf <= wf - 1))
        xc = jnp.clip(xf, 0, wf - 1).astype(jnp.int32)
        yc = jnp.clip(yf, 0, wf - 1).astype(jnp.int32)
        pix = start + yc * wi + xc
        row = base + pix * H + h_idx
        wgt = attn * wxy * jnp.where(valid, 1.0, 0.0)
        idx_ref[0, ci] = row
        wt_ref[0, ci] = wgt


def _prep(queriesT, rpT, w_qp, b_qp):
    return pl.pallas_call(
        _prep_body,
        grid=(B,),
        in_specs=[
            pl.BlockSpec((1, EMB, N), lambda b: (b, 0, 0)),
            pl.BlockSpec((1, 2, N), lambda b: (b, 0, 0)),
            pl.BlockSpec((QOUT, EMB), lambda b: (0, 0)),
            pl.BlockSpec((QOUT, 1), lambda b: (0, 0)),
        ],
        out_specs=[
            pl.BlockSpec((1, 4, 128, N), lambda b: (b, 0, 0, 0)),
            pl.BlockSpec((1, 4, 128, N), lambda b: (b, 0, 0, 0)),
        ],
        out_shape=[
            jax.ShapeDtypeStruct((B, 4, 128, N), jnp.int32),
            jax.ShapeDtypeStruct((B, 4, 128, N), jnp.float32),
        ],
    )(queriesT, rpT, w_qp, b_qp)


def _sc_gather_combine(idxq, wtq, table):
    mesh = plsc.VectorSubcoreMesh(core_axis_name="c", subcore_axis_name="s")

    @functools.partial(
        pl.kernel,
        mesh=mesh,
        out_type=jax.ShapeDtypeStruct((NQ, EMB), jnp.float32),
        compiler_params=pltpu.CompilerParams(use_tc_tiling_on_sc=False),
        scratch_types=[
            pltpu.VMEM((4, 128), jnp.int32),
            pltpu.VMEM((4, 128), jnp.float32),
            pltpu.VMEM((ROWS_Q, HEAD_DIM), jnp.float32),
            pltpu.VMEM((EMB,), jnp.float32),
            pltpu.SemaphoreType.DMA,
        ],
    )
    def k(idx_h, wt_h, tab_h, out_h, idx_v, wt_v, rows_v, out_v, sem):
        wid = lax.axis_index("s") * 2 + lax.axis_index("c")

        def body(t, carry):
            qi = t * NW + wid

            @pl.when(qi < NQ)
            def _():
                pltpu.sync_copy(idx_h.at[qi], idx_v)
                pltpu.sync_copy(wt_h.at[qi], wt_v)
                cps = [
                    pltpu.async_copy(tab_h.at[idx_v.at[ci]],
                                     rows_v.at[pl.ds(ci * 128, 128)], sem)
                    for ci in range(4)
                ]
                for cp in cps:
                    cp.wait()

                def hbody(h, c2):
                    acc0 = jnp.zeros((16,), jnp.float32)
                    acc1 = jnp.zeros((16,), jnp.float32)
                    for ci in range(4):
                        wrow = wt_v[ci, pl.ds(h * 16, 16)]
                        for lp in range(16):
                            slot = ci * 128 + h * 16 + lp
                            wgt = wrow[lp]
                            acc0 = acc0 + wgt * rows_v[slot, 0:16]
                            acc1 = acc1 + wgt * rows_v[slot, 16:32]
                    out_v[pl.ds(h * 32, 16)] = acc0
                    out_v[pl.ds(h * 32 + 16, 16)] = acc1
                    return c2

                lax.fori_loop(0, H, hbody, 0)
                pltpu.sync_copy(out_v, out_h.at[qi])

            return carry

        lax.fori_loop(0, Q_PER_W, body, 0)

    return k(idxq, wtq, table)


def _outproj_body(h_ref, w_ref, b_ref, o_ref):
    acc = jax.lax.dot_general(
        h_ref[0], w_ref[...], (((1,), (0,)), ((), ())),
        preferred_element_type=jnp.float32,
        precision=jax.lax.Precision.HIGHEST)
    o_ref[...] = (acc + b_ref[...])[None]


def _outproj(heads, w_t, bias2d):
    return pl.pallas_call(
        _outproj_body,
        grid=(B,),
        in_specs=[
            pl.BlockSpec((1, N, EMB), lambda b: (b, 0, 0)),
            pl.BlockSpec((EMB, EMB), lambda b: (0, 0)),
            pl.BlockSpec((1, EMB), lambda b: (0, 0)),
        ],
        out_specs=pl.BlockSpec((1, N, EMB), lambda b: (b, 0, 0)),
        out_shape=jax.ShapeDtypeStruct((B, N, EMB), jnp.float32),
    )(heads, w_t, bias2d)


_HLP = np.arange(H * L * P)
_WQ_ORDER = np.concatenate([_HLP * 3 + c for c in range(3)])


def kernel(img, img_shapes, queries, reference_points,
           W_img, b_img, W_q, b_q, W_out, b_out):
    x = _xproj(img, W_img.T, b_img[None])
    w_qp = jnp.take(W_q, _WQ_ORDER, axis=0)
    b_qp = jnp.take(b_q, _WQ_ORDER)[:, None]
    queriesT = jnp.transpose(queries, (0, 2, 1))
    rpT = jnp.transpose(reference_points, (0, 2, 1))
    idx4, wt4 = _prep(queriesT, rpT, w_qp, b_qp)
    idxq = jnp.transpose(idx4, (0, 3, 1, 2)).reshape(NQ, 4, 128)
    wtq = jnp.transpose(wt4, (0, 3, 1, 2)).reshape(NQ, 4, 128)
    table = x.reshape(B * I_TOT * H, HEAD_DIM)
    heads = _sc_gather_combine(idxq, wtq, table)
    return _outproj(heads.reshape(B, N, EMB), W_out.T, b_out[None])


# speedup vs baseline: 71.2264x; 1.5486x over previous
import functools

import numpy as np
import jax
import jax.numpy as jnp
from jax import lax
from jax.experimental import pallas as pl
from jax.experimental.pallas import tpu as pltpu
from jax.experimental.pallas import tpu_sc as plsc

B = 4
N = 900
I_TOT = 5440
H = 8
L = 4
P = 4
HEAD_DIM = 32
EMB = 256
QOUT = 384
NQ = B * N
ROWS_Q = 4 * H * L * P

NW = 32
Q_PER_W = -(-NQ // NW)

_LEVEL_W = (64.0, 32.0, 16.0, 8.0)
_LEVEL_START = (0, 4096, 5120, 5376)


def _xproj_body(img_ref, w_ref, b_ref, o_ref):
    acc = jax.lax.dot_general(
        img_ref[0], w_ref[...], (((1,), (0,)), ((), ())),
        preferred_element_type=jnp.float32,
        precision=jax.lax.Precision.HIGHEST)
    o_ref[...] = (acc + b_ref[...])[None]


def _xproj(img, w_t, bias2d):
    tile = 680
    return pl.pallas_call(
        _xproj_body,
        grid=(B, I_TOT // tile),
        in_specs=[
            pl.BlockSpec((1, tile, EMB), lambda b, i: (b, i, 0)),
            pl.BlockSpec((EMB, EMB), lambda b, i: (0, 0)),
            pl.BlockSpec((1, EMB), lambda b, i: (0, 0)),
        ],
        out_specs=pl.BlockSpec((1, tile, EMB), lambda b, i: (b, i, 0)),
        out_shape=jax.ShapeDtypeStruct((B, I_TOT, EMB), jnp.float32),
    )(img, w_t, bias2d)


def _prep_body(qT_ref, rpT_ref, w_ref, b_ref, idx_ref, wt_ref):
    b = pl.program_id(0)
    q = jax.lax.dot_general(
        w_ref[...], qT_ref[0], (((1,), (0,)), ((), ())),
        preferred_element_type=jnp.float32,
        precision=jax.lax.Precision.HIGHEST) + b_ref[...]
    xo = q[0:128]
    yo = q[128:256]
    lg = q[256:384]
    attn_blocks = []
    for h in range(H):
        blk = lg[h * 16:(h + 1) * 16]
        m = jnp.max(blk, axis=0, keepdims=True)
        e = jnp.exp(blk - m)
        s = jnp.sum(e, axis=0, keepdims=True)
        attn_blocks.append(e / s)
    attn = jnp.concatenate(attn_blocks, axis=0)

    rx = rpT_ref[0, 0:1]
    ry = rpT_ref[0, 1:2]

    r_iota = lax.broadcasted_iota(jnp.int32, (128, 900), 0)
    li = (r_iota // 4) % 4
    h_idx = r_iota // 16
    wf = jnp.where(li == 0, _LEVEL_W[0],
                   jnp.where(li == 1, _LEVEL_W[1],
                             jnp.where(li == 2, _LEVEL_W[2], _LEVEL_W[3])))
    wi = jnp.where(li == 0, 64, jnp.where(li == 1, 32,
                                          jnp.where(li == 2, 16, 8)))
    start = jnp.where(li == 0, _LEVEL_START[0],
                      jnp.where(li == 1, _LEVEL_START[1],
                                jnp.where(li == 2, _LEVEL_START[2],
                                          _LEVEL_START[3])))

    ix = rx * wf + xo - 0.5
    iy = ry * wf + yo - 0.5
    x0 = jnp.floor(ix)
    y0 = jnp.floor(iy)
    wx1 = ix - x0
    wx0 = 1.0 - wx1
    wy1 = iy - y0
    wy0 = 1.0 - wy1

    base = b * (I_TOT * H)
    for ci, (cx, cy) in enumerate(((0, 0), (1, 0), (0, 1), (1, 1))):
        xf = x0 + cx
        yf = y0 + cy
        wxy = (wx1 if cx else wx0) * (wy1 if cy else wy0)
        valid = ((xf >= 0) & (xf <= wf - 1) & (yf >= 0) & (yf <= wf - 1))
        xc = jnp.clip(xf, 0, wf - 1).astype(jnp.int32)
        yc = jnp.clip(yf, 0, wf - 1).astype(jnp.int32)
        pix = start + yc * wi + xc
        row = base + pix * H + h_idx
        wgt = attn * wxy * jnp.where(valid, 1.0, 0.0)
        idx_ref[0, ci] = row
        wt_ref[0, ci] = wgt


def _prep(queriesT, rpT, w_qp, b_qp):
    return pl.pallas_call(
        _prep_body,
        grid=(B,),
        in_specs=[
            pl.BlockSpec((1, EMB, N), lambda b: (b, 0, 0)),
            pl.BlockSpec((1, 2, N), lambda b: (b, 0, 0)),
            pl.BlockSpec((QOUT, EMB), lambda b: (0, 0)),
            pl.BlockSpec((QOUT, 1), lambda b: (0, 0)),
        ],
        out_specs=[
            pl.BlockSpec((1, 4, 128, N), lambda b: (b, 0, 0, 0)),
            pl.BlockSpec((1, 4, 128, N), lambda b: (b, 0, 0, 0)),
        ],
        out_shape=[
            jax.ShapeDtypeStruct((B, 4, 128, N), jnp.int32),
            jax.ShapeDtypeStruct((B, 4, 128, N), jnp.float32),
        ],
    )(queriesT, rpT, w_qp, b_qp)


def _sc_gather_combine(idxq, wtq, table):
    mesh = plsc.VectorSubcoreMesh(core_axis_name="c", subcore_axis_name="s")

    @functools.partial(
        pl.kernel,
        mesh=mesh,
        out_type=jax.ShapeDtypeStruct((NQ, EMB), jnp.float32),
        compiler_params=pltpu.CompilerParams(use_tc_tiling_on_sc=False),
        scratch_types=[
            pltpu.VMEM((2, 4, 128), jnp.int32),
            pltpu.VMEM((2, 4, 128), jnp.float32),
            pltpu.VMEM((2, ROWS_Q, HEAD_DIM), jnp.float32),
            pltpu.VMEM((2, EMB), jnp.float32),
            pltpu.SemaphoreType.DMA,
            pltpu.SemaphoreType.DMA,
            pltpu.SemaphoreType.DMA,
            pltpu.SemaphoreType.DMA,
            pltpu.SemaphoreType.DMA,
            pltpu.SemaphoreType.DMA,
        ],
    )
    def k(idx_h, wt_h, tab_h, out_h, idx_v, wt_v, rows_v, out_v,
          sf0, sf1, sg0, sg1, ss0, ss1):
        wid = lax.axis_index("s") * 2 + lax.axis_index("c")
        sf = (sf0, sf1)
        sg = (sg0, sg1)
        ss = (ss0, ss1)

        def fetch(t, p):
            @pl.when(t * NW + wid < NQ)
            def _():
                pltpu.async_copy(idx_h.at[t * NW + wid], idx_v.at[p], sf[p])
                pltpu.async_copy(wt_h.at[t * NW + wid], wt_v.at[p], sf[p])

        def wait_fetch(t, p):
            @pl.when(t * NW + wid < NQ)
            def _():
                pltpu.make_async_copy(idx_h.at[0], idx_v.at[p], sf[p]).wait()
                pltpu.make_async_copy(wt_h.at[0], wt_v.at[p], sf[p]).wait()

        def gathers(t, p):
            @pl.when(t * NW + wid < NQ)
            def _():
                for ci in range(4):
                    pltpu.async_copy(tab_h.at[idx_v.at[p, ci]],
                                     rows_v.at[p, pl.ds(ci * 128, 128)],
                                     sg[p])

        def wait_gathers(t, p):
            @pl.when(t * NW + wid < NQ)
            def _():
                pltpu.make_async_copy(tab_h.at[pl.ds(0, ROWS_Q)],
                                      rows_v.at[p], sg[p]).wait()

        def wait_store(t, p):
            @pl.when((t >= 0) & (t * NW + wid < NQ))
            def _():
                pltpu.make_async_copy(out_v.at[p], out_h.at[0], ss[p]).wait()

        def compute(t, p):
            qi = t * NW + wid

            @pl.when(qi < NQ)
            def _():
                def hbody(h, c2):
                    acc0 = jnp.zeros((16,), jnp.float32)
                    acc1 = jnp.zeros((16,), jnp.float32)
                    for ci in range(4):
                        wrow = wt_v[p, ci, pl.ds(h * 16, 16)]
                        for lp in range(16):
                            slot = ci * 128 + h * 16 + lp
                            wgt = wrow[lp]
                            acc0 = acc0 + wgt * rows_v[p, slot, 0:16]
                            acc1 = acc1 + wgt * rows_v[p, slot, 16:32]
                    out_v[p, pl.ds(h * 32, 16)] = acc0
                    out_v[p, pl.ds(h * 32 + 16, 16)] = acc1
                    return c2

                lax.fori_loop(0, H, hbody, 0)
                pltpu.async_copy(out_v.at[p], out_h.at[qi], ss[p])

        fetch(0, 0)
        wait_fetch(0, 0)
        gathers(0, 0)
        fetch(1, 1)

        def body(kk, carry):
            for u in range(2):
                t = 2 * kk + u
                p = u
                wait_gathers(t, p)
                wait_fetch(t + 1, 1 - p)
                gathers(t + 1, 1 - p)
                wait_store(t - 2, p)
                compute(t, p)
                fetch(t + 2, p)
            return carry

        lax.fori_loop(0, (Q_PER_W + 1) // 2, body, 0)
        wait_store(2 * ((Q_PER_W + 1) // 2) - 2, 0)

    return k(idxq, wtq, table)


def _outproj_body(h_ref, w_ref, b_ref, o_ref):
    acc = jax.lax.dot_general(
        h_ref[0], w_ref[...], (((1,), (0,)), ((), ())),
        preferred_element_type=jnp.float32,
        precision=jax.lax.Precision.HIGHEST)
    o_ref[...] = (acc + b_ref[...])[None]


def _outproj(heads, w_t, bias2d):
    return pl.pallas_call(
        _outproj_body,
        grid=(B,),
        in_specs=[
            pl.BlockSpec((1, N, EMB), lambda b: (b, 0, 0)),
            pl.BlockSpec((EMB, EMB), lambda b: (0, 0)),
            pl.BlockSpec((1, EMB), lambda b: (0, 0)),
        ],
        out_specs=pl.BlockSpec((1, N, EMB), lambda b: (b, 0, 0)),
        out_shape=jax.ShapeDtypeStruct((B, N, EMB), jnp.float32),
    )(heads, w_t, bias2d)


_HLP = np.arange(H * L * P)
_WQ_ORDER = np.concatenate([_HLP * 3 + c for c in range(3)])


def kernel(img, img_shapes, queries, reference_points,
           W_img, b_img, W_q, b_q, W_out, b_out):
    x = _xproj(img, W_img.T, b_img[None])
    w_qp = jnp.take(W_q, _WQ_ORDER, axis=0)
    b_qp = jnp.take(b_q, _WQ_ORDER)[:, None]
    queriesT = jnp.transpose(queries, (0, 2, 1))
    rpT = jnp.transpose(reference_points, (0, 2, 1))
    idx4, wt4 = _prep(queriesT, rpT, w_qp, b_qp)
    idxq = jnp.transpose(idx4, (0, 3, 1, 2)).reshape(NQ, 4, 128)
    wtq = jnp.transpose(wt4, (0, 3, 1, 2)).reshape(NQ, 4, 128)
    table = x.reshape(B * I_TOT * H, HEAD_DIM)
    heads = _sc_gather_combine(idxq, wtq, table)
    return _outproj(heads.reshape(B, N, EMB), W_out.T, b_out[None])
